# Initial kernel scaffold; baseline (speedup 1.0000x reference)
import jax
import jax.numpy as jnp
from jax.experimental import pallas as pl


def kernel(features, labels, head_w):
    raise NotImplementedError("write your pallas kernel here")


# speedup vs baseline: 13.0823x; 13.0823x over previous
import functools

import jax
import jax.numpy as jnp
from jax import lax
from jax.experimental import pallas as pl
from jax.experimental.pallas import tpu as pltpu

K = 2
LAMBDA_MAX = 0.1
DELTA = 0.5


def _bag_kernel(f_ref, w_ref, out_ref):
    b = pl.program_id(0)
    n = f_ref.shape[1]

    f = f_ref[0]
    w = w_ref[...]

    a = jnp.dot(f, w, preferred_element_type=jnp.float32)
    sq = jnp.sum(f * f, axis=1, keepdims=True)

    iota = lax.broadcasted_iota(jnp.int32, (n, 1), 0)
    neg_inf = jnp.float32(-jnp.inf)

    m1 = jnp.max(a)
    i1 = jnp.min(jnp.where(a == m1, iota, n))
    a_m1 = jnp.where(iota == i1, neg_inf, a)
    m2 = jnp.max(a_m1)
    i2 = jnp.min(jnp.where(a_m1 == m2, iota, n))

    f1 = f_ref[0, pl.ds(i1, 1), :]
    f2 = f_ref[0, pl.ds(i2, 1), :]
    mu = (f1 + f2) * 0.5
    mu_n = mu / jnp.maximum(jnp.sqrt(jnp.sum(mu * mu)), 1e-12)

    dots = jnp.sum(f * mu_n, axis=1, keepdims=True)
    inv_norm = 1.0 / jnp.maximum(jnp.sqrt(sq), 1e-12)
    d = 1.0 - dots * inv_norm

    key_mask = (iota == i1) | (iota == i2)
    d_o = jnp.where(key_mask, neg_inf, d)
    loss_max = jnp.maximum(jnp.max(d_o) - DELTA, 0.0)

    a_o = jnp.where(key_mask, neg_inf, a)
    c = jnp.max(a_o)
    e = jnp.where(key_mask, 0.0, jnp.exp(a_o - c))
    s_e = jnp.sum(e)
    s_ed = jnp.sum(e * d)
    loss = s_ed / s_e + LAMBDA_MAX * loss_max

    @pl.when(b == 0)
    def _():
        out_ref[0, 0] = 0.0

    out_ref[0, 0] += loss


def kernel(features, labels, head_w):
    del labels
    b, n, d = features.shape
    total = pl.pallas_call(
        _bag_kernel,
        grid=(b,),
        in_specs=[
            pl.BlockSpec((1, n, d), lambda i: (i, 0, 0)),
            pl.BlockSpec((d, 1), lambda i: (0, 0)),
        ],
        out_specs=pl.BlockSpec(
            (1, 1), lambda i: (0, 0), memory_space=pltpu.SMEM
        ),
        out_shape=jax.ShapeDtypeStruct((1, 1), jnp.float32),
    )(features, head_w)
    return total[0, 0] / b
